# Initial kernel scaffold; baseline (speedup 1.0000x reference)
import jax
import jax.numpy as jnp
from jax.experimental import pallas as pl


def kernel(img, targets, slots, W_enc, b_enc, W_prev, b_prev, codebook, W_post, b_post, W_dec, b_dec):
    raise NotImplementedError("write your pallas kernel here")


# speedup vs baseline: 2.6043x; 2.6043x over previous
import jax
import jax.numpy as jnp
from jax.experimental import pallas as pl

_N_SLOTS = 64
_EMBED_DIM = 256
_N_CODES = 8192
_BETA = 0.25


def _fused_vq_kernel(slots_ref, W_enc_ref, b_enc_ref, W_prev_ref, b_prev_ref,
                     cb_ref, W_post_ref, b_post_ref, W_dec_ref, b_dec_ref,
                     rec_ref, loss_ref, idx_ref):
    f32 = jnp.float32
    h = jnp.maximum(
        jnp.dot(slots_ref[...], W_enc_ref[...], preferred_element_type=f32)
        + b_enc_ref[...], 0.0)
    s = (jnp.dot(h, W_prev_ref[...], preferred_element_type=f32)
         + b_prev_ref[...])
    cb = cb_ref[...]
    a = jnp.sum(s * s, axis=1, keepdims=True)
    m = jax.lax.dot_general(s, cb, (((1,), (1,)), ((), ())),
                            preferred_element_type=f32)
    cn = jnp.sum(cb * cb, axis=1)
    d = (a - 2.0 * m) + cn[None, :]
    dmin = jnp.min(d, axis=1, keepdims=True)
    col = jax.lax.broadcasted_iota(jnp.int32, d.shape, 1)
    big = jnp.int32(jnp.iinfo(jnp.int32).max)
    idx = jnp.min(jnp.where(d == dmin, col, big), axis=1)
    onehot = (col == idx[:, None]).astype(f32)
    zq = jnp.dot(onehot, cb, preferred_element_type=f32)
    diff = zq - s
    loss = (1.0 + _BETA) * jnp.sum(diff * diff) / (_N_SLOTS * _EMBED_DIM)
    loss_ref[...] = jnp.reshape(loss, (1, 1))
    dec_in = (jnp.dot(zq, W_post_ref[...], preferred_element_type=f32)
              + b_post_ref[...])
    rec = (jnp.dot(dec_in, W_dec_ref[...], preferred_element_type=f32)
           + b_dec_ref[...])
    rec_ref[...] = jnp.clip(rec, -1.0, 1.0)
    idx_ref[...] = jnp.reshape(idx, (1, _N_SLOTS))


def kernel(img, targets, slots, W_enc, b_enc, W_prev, b_prev, codebook,
           W_post, b_post, W_dec, b_dec):
    bs = img.shape[0]
    enc_dim = W_dec.shape[1]
    rec1, loss, idx = pl.pallas_call(
        _fused_vq_kernel,
        out_shape=[
            jax.ShapeDtypeStruct((_N_SLOTS, enc_dim), jnp.float32),
            jax.ShapeDtypeStruct((1, 1), jnp.float32),
            jax.ShapeDtypeStruct((1, _N_SLOTS), jnp.int32),
        ],
    )(slots, W_enc, b_enc.reshape(1, -1), W_prev, b_prev.reshape(1, -1),
      codebook, W_post, b_post.reshape(1, -1), W_dec, b_dec.reshape(1, -1))
    rec = jnp.broadcast_to(rec1[None], (bs, _N_SLOTS, enc_dim))
    q_indices = jnp.broadcast_to(idx, (bs, _N_SLOTS))
    return rec, jnp.reshape(loss, ()), q_indices
